# baseline (speedup 1.0000x reference)
import math

import jax
import jax.numpy as jnp
from jax import lax
from jax.experimental import pallas as pl
from jax.experimental.pallas import tpu as pltpu
from jax.experimental.pallas import tpu_sc as plsc

_NUM_LEVELS = 16
_F = 2
_T = 2 ** 19
_MASK = _T - 1
_BASE_RES = 16
_FINEST_RES = 2048
_GROWTH = math.exp(
    (math.log(_FINEST_RES) - math.log(_BASE_RES)) / (_NUM_LEVELS - 1))
_RES = [max(1, int(round(_BASE_RES * _GROWTH ** l)))
        for l in range(_NUM_LEVELS)]
_N = 262144
_NC, _NS = 2, 16
_NW = _NC * _NS
_PW = _N // _NW
_C = 128
_NCHUNK = _PW // _C
_G = _C // 16
_PA = 73856093
_PB = 19349663
_PC = 83492791


def _body(x_hbm, tab_hbm, resb_hbm, out_hbm, sh0, sh1, x_all,
          idx0, idx1, w0v, w1v, fW0, fW1, res0, res1, resb_v,
          sem0, sem1, semr0, semr1, sems):
    cid = lax.convert_element_type(lax.axis_index("c"), jnp.int32)
    sid = lax.convert_element_type(lax.axis_index("s"), jnp.int32)
    wid32 = sid * jnp.int32(_NC) + cid
    idx_b = (idx0, idx1)
    w_b = (w0v, w1v)
    fW_b = (fW0, fW1)
    res_b = (res0, res1)
    sem_b = (sem0, sem1)
    semr_b = (semr0, semr1)
    sh_b = (sh0, sh1)
    wblock = wid32 * jnp.int32(_NCHUNK)
    stage_per = _T // _NS

    pltpu.sync_copy(x_hbm.at[pl.ds(wblock * jnp.int32(384), _PW * 3)], x_all)
    pltpu.sync_copy(resb_hbm, resb_v)

    def pass1(p, cl, res, sh):
        idx_v, w_v = idx_b[p], w_b[p]
        base = cl * jnp.int32(384)
        for g in range(_G):
            xs0 = x_all[pl.ds(base + jnp.int32(g * 16), 16)] * res
            xs1 = x_all[pl.ds(base + jnp.int32(g * 16 + 128), 16)] * res
            xs2 = x_all[pl.ds(base + jnp.int32(g * 16 + 256), 16)] * res
            i0 = xs0.astype(jnp.int32)
            i1 = xs1.astype(jnp.int32)
            i2 = xs2.astype(jnp.int32)
            w_v[pl.ds(jnp.int32(g * 16), 16)] = \
                xs0 - i0.astype(jnp.float32)
            w_v[pl.ds(jnp.int32(g * 16 + 128), 16)] = \
                xs1 - i1.astype(jnp.float32)
            w_v[pl.ds(jnp.int32(g * 16 + 256), 16)] = \
                xs2 - i2.astype(jnp.float32)
            a0 = i0 * jnp.int32(_PA)
            a1 = a0 + jnp.int32(_PA)
            b0 = i1 * jnp.int32(_PB)
            b1 = b0 + jnp.int32(_PB)
            c0 = i2 * jnp.int32(_PC)
            c1 = c0 + jnp.int32(_PC)
            corner = 0
            for aa in (a0, a1):
                for bb in (b0, b1):
                    for cc in (c0, c1):
                        h = (aa ^ bb ^ cc) & jnp.int32(_MASK)
                        idx_v[jnp.int32(corner),
                              pl.ds(jnp.int32(g * 16), 16)] = h
                        corner += 1
        for c in range(8):
            pltpu.async_copy(sh.at[idx_v.at[jnp.int32(c)]],
                             fW_b[p].at[pl.ds(jnp.int32(c * _C), _C)],
                             sem_b[p])

    def drain(p):
        pltpu.make_async_copy(tab_hbm.at[pl.ds(0, 8 * _C)], fW_b[p],
                              sem_b[p]).wait()

    def combine(p, cl, l, not_first):
        w_v, fW, res_v = w_b[p], fW_b[p], res_b[p]

        @pl.when(not_first)
        def _():
            pltpu.make_async_copy(res_v, out_hbm.at[pl.ds(0, 2 * _C)],
                                  semr_b[p]).wait()

        for g in range(_G):
            s = jnp.int32(g * 16)
            w0 = w_v[pl.ds(s, 16)]
            w1 = w_v[pl.ds(jnp.int32(g * 16 + 128), 16)]
            w2 = w_v[pl.ds(jnp.int32(g * 16 + 256), 16)]
            u0 = 1.0 - w0
            u1 = 1.0 - w1
            u2 = 1.0 - w2
            p00 = u0 * u1
            p01 = u0 * w1
            p10 = w0 * u1
            p11 = w0 * w1
            wts = (p00 * u2, p00 * w2, p01 * u2, p01 * w2,
                   p10 * u2, p10 * w2, p11 * u2, p11 * w2)
            acc0 = None
            acc1 = None
            for c in range(8):
                wv = fW[pl.ds(jnp.int32(c * _C + g * 16), 16)]
                f0 = lax.bitcast_convert_type(
                    lax.shift_left(wv, jnp.uint32(16)), jnp.float32)
                f1 = lax.bitcast_convert_type(
                    wv & jnp.uint32(0xFFFF0000), jnp.float32)
                t0 = wts[c] * f0
                t1 = wts[c] * f1
                acc0 = t0 if acc0 is None else acc0 + t0
                acc1 = t1 if acc1 is None else acc1 + t1
            res_v[pl.ds(s, 16)] = acc0
            res_v[pl.ds(jnp.int32(g * 16 + 128), 16)] = acc1
        pltpu.async_copy(
            res_v,
            out_hbm.at[pl.ds(l * jnp.int32(2 * _N)
                             + (wblock + cl) * jnp.int32(256), 256)],
            semr_b[p])

    def stage_fire(B, l):
        pltpu.async_copy(
            tab_hbm.at[pl.ds(l * jnp.int32(_T)
                             + sid * jnp.int32(stage_per), stage_per)],
            sh_b[B].at[pl.ds(sid * jnp.int32(stage_per), stage_per)],
            sems)

    def stage_wait(B):
        pltpu.make_async_copy(
            tab_hbm.at[pl.ds(0, stage_per)],
            sh_b[B].at[pl.ds(sid * jnp.int32(stage_per), stage_per)],
            sems).wait()

    stage_fire(0, jnp.int32(0))

    def dlevel_body(d, carry0):
        for m in (0, 1):
            l = d * jnp.int32(2) + jnp.int32(m)
            sh = sh_b[m]
            res = resb_v[pl.ds(l * jnp.int32(16), 16)]
            stage_wait(m)
            plsc.subcore_barrier()

            @pl.when(l < jnp.int32(_NUM_LEVELS - 1))
            def _():
                stage_fire(m ^ 1, l + jnp.int32(1))

            pass1(0, jnp.int32(0), res, sh)

            def pair_body(j, carry, l=l, res=res, sh=sh):
                cl0 = j * jnp.int32(2)
                nf = (l > jnp.int32(0)) | (j > jnp.int32(0))
                pass1(1, cl0 + jnp.int32(1), res, sh)
                drain(0)
                combine(0, cl0, l, nf)

                @pl.when(j < jnp.int32(_NCHUNK // 2 - 1))
                def _():
                    pass1(0, cl0 + jnp.int32(2), res, sh)

                drain(1)
                combine(1, cl0 + jnp.int32(1), l, nf)
                return carry

            lax.fori_loop(jnp.int32(0), jnp.int32(_NCHUNK // 2), pair_body,
                          jnp.int32(0))
        return carry0

    lax.fori_loop(jnp.int32(0), jnp.int32(_NUM_LEVELS // 2), dlevel_body,
                  jnp.int32(0))

    for p in (0, 1):
        pltpu.make_async_copy(res_b[p], out_hbm.at[pl.ds(0, 2 * _C)],
                              semr_b[p]).wait()


def kernel(x, tables):
    mesh = plsc.VectorSubcoreMesh(
        core_axis_name="c", subcore_axis_name="s",
        num_cores=_NC, num_subcores=_NS)
    k = pl.kernel(
        _body,
        out_type=jax.ShapeDtypeStruct((_NUM_LEVELS * _N * _F,), jnp.float32),
        mesh=mesh,
        scratch_types=[
            pltpu.VMEM_SHARED((_T,), jnp.uint32),
            pltpu.VMEM_SHARED((_T,), jnp.uint32),
            pltpu.VMEM((_PW * 3,), jnp.float32),
            pltpu.VMEM((8, _C), jnp.int32),
            pltpu.VMEM((8, _C), jnp.int32),
            pltpu.VMEM((3 * _C,), jnp.float32),
            pltpu.VMEM((3 * _C,), jnp.float32),
            pltpu.VMEM((8 * _C,), jnp.uint32),
            pltpu.VMEM((8 * _C,), jnp.uint32),
            pltpu.VMEM((2 * _C,), jnp.float32),
            pltpu.VMEM((2 * _C,), jnp.float32),
            pltpu.VMEM((_NUM_LEVELS * 16,), jnp.float32),
            pltpu.SemaphoreType.DMA,
            pltpu.SemaphoreType.DMA,
            pltpu.SemaphoreType.DMA,
            pltpu.SemaphoreType.DMA,
            pltpu.SemaphoreType.DMA,
        ],
        compiler_params=pltpu.CompilerParams(
            needs_layout_passes=False, use_tc_tiling_on_sc=False),
    )
    xf = x.astype(jnp.float32)
    xin = xf.reshape(_N // _C, _C, 3).transpose(0, 2, 1).reshape(3 * _N)
    tphys = tables.astype(jnp.float32) \
        .reshape(_NUM_LEVELS, _T // _C, _C, _F).transpose(0, 1, 3, 2)
    b0 = lax.bitcast_convert_type(tphys[:, :, 0, :], jnp.uint32)
    b1 = lax.bitcast_convert_type(tphys[:, :, 1, :], jnp.uint32)

    def _rne(u):
        return (u + jnp.uint32(0x7FFF) + ((u >> 16) & jnp.uint32(1))) \
            & jnp.uint32(0xFFFF0000)

    tab = (_rne(b1) | (_rne(b0) >> 16)).reshape(_NUM_LEVELS * _T)
    resb = jnp.broadcast_to(
        jnp.asarray(_RES, jnp.float32)[:, None],
        (_NUM_LEVELS, 16)).reshape(_NUM_LEVELS * 16)
    out = k(xin, tab, resb)
    return out.reshape(_NUM_LEVELS, _N // _C, _F, _C) \
        .transpose(1, 3, 0, 2).reshape(_N, _NUM_LEVELS, _F)


# speedup vs baseline: 1.0245x; 1.0245x over previous
import math

import jax
import jax.numpy as jnp
from jax import lax
from jax.experimental import pallas as pl
from jax.experimental.pallas import tpu as pltpu
from jax.experimental.pallas import tpu_sc as plsc

_NUM_LEVELS = 16
_F = 2
_T = 2 ** 19
_MASK = _T - 1
_BASE_RES = 16
_FINEST_RES = 2048
_GROWTH = math.exp(
    (math.log(_FINEST_RES) - math.log(_BASE_RES)) / (_NUM_LEVELS - 1))
_RES = [max(1, int(round(_BASE_RES * _GROWTH ** l)))
        for l in range(_NUM_LEVELS)]
_N = 262144
_NC, _NS = 2, 16
_NW = _NC * _NS
_PW = _N // _NW
_C = 128
_NCHUNK = _PW // _C
_G = _C // 16
_PA = 73856093
_PB = 19349663
_PC = 83492791


def _body(x_hbm, tab_hbm, resb_hbm, out_hbm, sh0, sh1, x_all,
          idx0, idx1, w0v, w1v, fW0, fW1, res0, res1, resb_v,
          sem0, sem1, semr0, semr1, sems):
    cid = lax.convert_element_type(lax.axis_index("c"), jnp.int32)
    sid = lax.convert_element_type(lax.axis_index("s"), jnp.int32)
    wid32 = sid * jnp.int32(_NC) + cid
    idx_b = (idx0, idx1)
    w_b = (w0v, w1v)
    fW_b = (fW0, fW1)
    res_b = (res0, res1)
    sem_b = (sem0, sem1)
    semr_b = (semr0, semr1)
    sh_b = (sh0, sh1)
    wblock = wid32 * jnp.int32(_NCHUNK)
    stage_per = _T // _NS

    pltpu.sync_copy(x_hbm.at[pl.ds(wblock * jnp.int32(384), _PW * 3)], x_all)
    pltpu.sync_copy(resb_hbm, resb_v)

    def pass1(p, cl, res, sh):
        idx_v, w_v = idx_b[p], w_b[p]
        base = cl * jnp.int32(384)
        for g in range(_G):
            xs0 = x_all[pl.ds(base + jnp.int32(g * 16), 16)] * res
            xs1 = x_all[pl.ds(base + jnp.int32(g * 16 + 128), 16)] * res
            xs2 = x_all[pl.ds(base + jnp.int32(g * 16 + 256), 16)] * res
            i0 = xs0.astype(jnp.int32)
            i1 = xs1.astype(jnp.int32)
            i2 = xs2.astype(jnp.int32)
            w_v[pl.ds(jnp.int32(g * 16), 16)] = \
                xs0 - i0.astype(jnp.float32)
            w_v[pl.ds(jnp.int32(g * 16 + 128), 16)] = \
                xs1 - i1.astype(jnp.float32)
            w_v[pl.ds(jnp.int32(g * 16 + 256), 16)] = \
                xs2 - i2.astype(jnp.float32)
            a0 = i0 * jnp.int32(_PA)
            a1 = a0 + jnp.int32(_PA)
            b0 = i1 * jnp.int32(_PB)
            b1 = b0 + jnp.int32(_PB)
            c0 = i2 * jnp.int32(_PC)
            c1 = c0 + jnp.int32(_PC)
            corner = 0
            for aa in (a0, a1):
                for bb in (b0, b1):
                    for cc in (c0, c1):
                        h = (aa ^ bb ^ cc) & jnp.int32(_MASK)
                        idx_v[jnp.int32(corner),
                              pl.ds(jnp.int32(g * 16), 16)] = h
                        corner += 1
        for c in range(8):
            pltpu.async_copy(sh.at[idx_v.at[jnp.int32(c)]],
                             fW_b[p].at[jnp.int32(c)], sem_b[p])

    def drain(p):
        dummy = tab_hbm.at[pl.ds(0, _C)]
        for c in range(8):
            pltpu.make_async_copy(dummy, fW_b[p].at[jnp.int32(c)],
                                  sem_b[p]).wait()

    def combine(p, cl, l, not_first):
        w_v, fW, res_v = w_b[p], fW_b[p], res_b[p]

        @pl.when(not_first)
        def _():
            pltpu.make_async_copy(res_v, out_hbm.at[pl.ds(0, 2 * _C)],
                                  semr_b[p]).wait()

        for g in range(_G):
            s = jnp.int32(g * 16)
            w0 = w_v[pl.ds(s, 16)]
            w1 = w_v[pl.ds(jnp.int32(g * 16 + 128), 16)]
            w2 = w_v[pl.ds(jnp.int32(g * 16 + 256), 16)]
            u0 = 1.0 - w0
            u1 = 1.0 - w1
            u2 = 1.0 - w2
            p00 = u0 * u1
            p01 = u0 * w1
            p10 = w0 * u1
            p11 = w0 * w1
            wts = (p00 * u2, p00 * w2, p01 * u2, p01 * w2,
                   p10 * u2, p10 * w2, p11 * u2, p11 * w2)
            acc0 = None
            acc1 = None
            for c in range(8):
                wv = fW[jnp.int32(c), pl.ds(s, 16)]
                f0 = lax.bitcast_convert_type(
                    lax.shift_left(wv, jnp.uint32(16)), jnp.float32)
                f1 = lax.bitcast_convert_type(
                    wv & jnp.uint32(0xFFFF0000), jnp.float32)
                t0 = wts[c] * f0
                t1 = wts[c] * f1
                acc0 = t0 if acc0 is None else acc0 + t0
                acc1 = t1 if acc1 is None else acc1 + t1
            res_v[pl.ds(s, 16)] = acc0
            res_v[pl.ds(jnp.int32(g * 16 + 128), 16)] = acc1
        pltpu.async_copy(
            res_v,
            out_hbm.at[pl.ds(l * jnp.int32(2 * _N)
                             + (wblock + cl) * jnp.int32(256), 256)],
            semr_b[p])

    def stage_fire(B, l):
        pltpu.async_copy(
            tab_hbm.at[pl.ds(l * jnp.int32(_T)
                             + sid * jnp.int32(stage_per), stage_per)],
            sh_b[B].at[pl.ds(sid * jnp.int32(stage_per), stage_per)],
            sems)

    def stage_wait(B):
        pltpu.make_async_copy(
            tab_hbm.at[pl.ds(0, stage_per)],
            sh_b[B].at[pl.ds(sid * jnp.int32(stage_per), stage_per)],
            sems).wait()

    stage_fire(0, jnp.int32(0))

    def dlevel_body(d, carry0):
        for m in (0, 1):
            l = d * jnp.int32(2) + jnp.int32(m)
            sh = sh_b[m]
            res = resb_v[pl.ds(l * jnp.int32(16), 16)]
            stage_wait(m)
            plsc.subcore_barrier()

            @pl.when(l < jnp.int32(_NUM_LEVELS - 1))
            def _():
                stage_fire(m ^ 1, l + jnp.int32(1))

            pass1(0, jnp.int32(0), res, sh)

            def pair_body(j, carry, l=l, res=res, sh=sh):
                cl0 = j * jnp.int32(2)
                nf = (l > jnp.int32(0)) | (j > jnp.int32(0))
                pass1(1, cl0 + jnp.int32(1), res, sh)
                drain(0)
                combine(0, cl0, l, nf)

                @pl.when(j < jnp.int32(_NCHUNK // 2 - 1))
                def _():
                    pass1(0, cl0 + jnp.int32(2), res, sh)

                drain(1)
                combine(1, cl0 + jnp.int32(1), l, nf)
                return carry

            lax.fori_loop(jnp.int32(0), jnp.int32(_NCHUNK // 2), pair_body,
                          jnp.int32(0))
        return carry0

    lax.fori_loop(jnp.int32(0), jnp.int32(_NUM_LEVELS // 2), dlevel_body,
                  jnp.int32(0))

    for p in (0, 1):
        pltpu.make_async_copy(res_b[p], out_hbm.at[pl.ds(0, 2 * _C)],
                              semr_b[p]).wait()


def kernel(x, tables):
    mesh = plsc.VectorSubcoreMesh(
        core_axis_name="c", subcore_axis_name="s",
        num_cores=_NC, num_subcores=_NS)
    k = pl.kernel(
        _body,
        out_type=jax.ShapeDtypeStruct((_NUM_LEVELS * _N * _F,), jnp.float32),
        mesh=mesh,
        scratch_types=[
            pltpu.VMEM_SHARED((_T,), jnp.uint32),
            pltpu.VMEM_SHARED((_T,), jnp.uint32),
            pltpu.VMEM((_PW * 3,), jnp.float32),
            pltpu.VMEM((8, _C), jnp.int32),
            pltpu.VMEM((8, _C), jnp.int32),
            pltpu.VMEM((3 * _C,), jnp.float32),
            pltpu.VMEM((3 * _C,), jnp.float32),
            pltpu.VMEM((8, _C), jnp.uint32),
            pltpu.VMEM((8, _C), jnp.uint32),
            pltpu.VMEM((2 * _C,), jnp.float32),
            pltpu.VMEM((2 * _C,), jnp.float32),
            pltpu.VMEM((_NUM_LEVELS * 16,), jnp.float32),
            pltpu.SemaphoreType.DMA,
            pltpu.SemaphoreType.DMA,
            pltpu.SemaphoreType.DMA,
            pltpu.SemaphoreType.DMA,
            pltpu.SemaphoreType.DMA,
        ],
        compiler_params=pltpu.CompilerParams(
            needs_layout_passes=False, use_tc_tiling_on_sc=False),
    )
    xf = x.astype(jnp.float32)
    xin = xf.reshape(_N // _C, _C, 3).transpose(0, 2, 1).reshape(3 * _N)
    tphys = tables.astype(jnp.float32) \
        .reshape(_NUM_LEVELS, _T // _C, _C, _F).transpose(0, 1, 3, 2)
    b0 = lax.bitcast_convert_type(tphys[:, :, 0, :], jnp.uint32)
    b1 = lax.bitcast_convert_type(tphys[:, :, 1, :], jnp.uint32)

    def _rne(u):
        return (u + jnp.uint32(0x7FFF) + ((u >> 16) & jnp.uint32(1))) \
            & jnp.uint32(0xFFFF0000)

    tab = (_rne(b1) | (_rne(b0) >> 16)).reshape(_NUM_LEVELS * _T)
    resb = jnp.broadcast_to(
        jnp.asarray(_RES, jnp.float32)[:, None],
        (_NUM_LEVELS, 16)).reshape(_NUM_LEVELS * 16)
    out = k(xin, tab, resb)
    return out.reshape(_NUM_LEVELS, _N // _C, _F, _C) \
        .transpose(1, 3, 0, 2).reshape(_N, _NUM_LEVELS, _F)
